# baseline (speedup 1.0000x reference)
import functools

import jax
import jax.numpy as jnp
from jax import lax
from jax.experimental import pallas as pl
from jax.experimental.pallas import tpu as pltpu
from jax.experimental.pallas import tpu_sc as plsc

N_ATOMS = 1_000_000
DIM = 128
CHUNK = 128
DCHUNK = 2 * CHUNK
N_FULL = N_ATOMS // CHUNK
TAIL = N_ATOMS - N_FULL * CHUNK
N_Z = 119
NC = 2
NS = 16
NW = NC * NS
BASE_CHUNKS = N_FULL // NW
EXTRA = N_FULL - BASE_CHUNKS * NW
N_DBL = BASE_CHUNKS // 2
NBUF = 3
ROUNDS = -(-N_DBL // NBUF)
SLAB = (BASE_CHUNKS + 1) * CHUNK

_mesh = plsc.VectorSubcoreMesh(core_axis_name="c", subcore_axis_name="s")


@functools.partial(
    pl.kernel,
    mesh=_mesh,
    out_type=jax.ShapeDtypeStruct((N_ATOMS, DIM), jnp.float32),
    scratch_types=[
        pltpu.VMEM((SLAB,), jnp.int32),
        pltpu.VMEM((NBUF, DCHUNK, DIM), jnp.float32),
        pltpu.VMEM_SHARED((N_Z, DIM), jnp.float32),
        pltpu.SemaphoreType.DMA,
    ]
    + [pltpu.SemaphoreType.DMA] * NBUF
    + [pltpu.SemaphoreType.DMA] * NBUF,
)
def _embed(idx_hbm, table_hbm, out_hbm, idx_v, rows_v, table_sh, sem, *bsems):
    gsem = bsems[:NBUF]
    ssem = bsems[NBUF:]
    sid = lax.axis_index("s")
    wid = sid * NC + lax.axis_index("c")
    start_chunk = wid * BASE_CHUNKS + jnp.minimum(wid, EXTRA)
    atom0 = start_chunk * CHUNK

    @pl.when(sid == 0)
    def _stage_table():
        pltpu.sync_copy(table_hbm, table_sh)

    pltpu.sync_copy(
        idx_hbm.at[pl.ds(atom0, BASE_CHUNKS * CHUNK)],
        idx_v.at[pl.ds(0, BASE_CHUNKS * CHUNK)],
    )

    @pl.when(wid < EXTRA)
    def _extra_idx():
        pltpu.sync_copy(
            idx_hbm.at[pl.ds(atom0 + BASE_CHUNKS * CHUNK, CHUNK)],
            idx_v.at[pl.ds(BASE_CHUNKS * CHUNK, CHUNK)],
        )

    @pl.when(wid == NW - 1)
    def _tail_idx():
        pltpu.sync_copy(
            idx_hbm.at[pl.ds(N_FULL * CHUNK, TAIL)],
            idx_v.at[pl.ds(BASE_CHUNKS * CHUNK, TAIL)],
        )

    plsc.subcore_barrier()

    def _gather_half(d, b, h):
        return pltpu.make_async_copy(
            table_sh.at[idx_v.at[pl.ds((2 * d + h) * CHUNK, CHUNK)]],
            rows_v.at[b].at[pl.ds(h * CHUNK, CHUNK)],
            gsem[b],
        )

    def _store_copy(d, b):
        return pltpu.make_async_copy(
            rows_v.at[b],
            out_hbm.at[pl.ds(atom0 + d * DCHUNK, DCHUNK)],
            ssem[b],
        )

    def round_body(r, carry):
        for b in range(NBUF):
            d = r * NBUF + b

            @pl.when(d < N_DBL)
            def _(b=b, d=d):
                @pl.when(r >= 1)
                def _wait_prev():
                    _store_copy(d - NBUF, b).wait()

                _gather_half(d, b, 0).start()
                _gather_half(d, b, 1).start()

        for b in range(NBUF):
            d = r * NBUF + b

            @pl.when(d < N_DBL)
            def _(b=b, d=d):
                _gather_half(d, b, 0).wait()
                _gather_half(d, b, 1).wait()
                _store_copy(d, b).start()

        return carry

    lax.fori_loop(0, ROUNDS, round_body, 0)

    for b in range(NBUF):
        d = (ROUNDS - 1) * NBUF + b
        dlast = d if d < N_DBL else d - NBUF
        _store_copy(dlast, b).wait()

    @pl.when(wid < EXTRA)
    def _odd_chunk():
        idx_o = idx_v.at[pl.ds(BASE_CHUNKS * CHUNK, CHUNK)]
        rows_o = rows_v.at[0].at[pl.ds(0, CHUNK)]
        pltpu.async_copy(table_sh.at[idx_o], rows_o, sem).wait()
        pltpu.sync_copy(
            rows_o, out_hbm.at[pl.ds(atom0 + BASE_CHUNKS * CHUNK, CHUNK)]
        )

    @pl.when(wid == NW - 1)
    def _tail():
        idx_t = idx_v.at[pl.ds(BASE_CHUNKS * CHUNK, TAIL)]
        rows_t = rows_v.at[0].at[pl.ds(0, TAIL)]
        pltpu.async_copy(table_sh.at[idx_t], rows_t, sem).wait()
        pltpu.sync_copy(rows_t, out_hbm.at[pl.ds(N_FULL * CHUNK, TAIL)])


def kernel(Z, embeddings):
    return _embed(Z.astype(jnp.int32), embeddings)


# speedup vs baseline: 1.0410x; 1.0410x over previous
import functools

import jax
import jax.numpy as jnp
from jax import lax
from jax.experimental import pallas as pl
from jax.experimental.pallas import tpu as pltpu
from jax.experimental.pallas import tpu_sc as plsc

N_ATOMS = 1_000_000
DIM = 128
CHUNK = 128
N_FULL = N_ATOMS // CHUNK
TAIL = N_ATOMS - N_FULL * CHUNK
N_Z = 119
NC = 2
NS = 16
NW = NC * NS
BASE_CHUNKS = N_FULL // NW
EXTRA = N_FULL - BASE_CHUNKS * NW
NBUF = 6
ROUNDS = -(-(BASE_CHUNKS + 1) // NBUF)
SLAB = (BASE_CHUNKS + 1) * CHUNK

_mesh = plsc.VectorSubcoreMesh(core_axis_name="c", subcore_axis_name="s")


@functools.partial(
    pl.kernel,
    mesh=_mesh,
    out_type=jax.ShapeDtypeStruct((N_ATOMS, DIM), jnp.float32),
    scratch_types=[
        pltpu.VMEM((SLAB,), jnp.int32),
        pltpu.VMEM((NBUF, CHUNK, DIM), jnp.float32),
        pltpu.VMEM_SHARED((N_Z, DIM), jnp.float32),
        pltpu.SemaphoreType.DMA,
    ]
    + [pltpu.SemaphoreType.DMA] * NBUF
    + [pltpu.SemaphoreType.DMA] * NBUF,
)
def _embed(idx_hbm, table_hbm, out_hbm, idx_v, rows_v, table_sh, sem, *bsems):
    gsem = bsems[:NBUF]
    ssem = bsems[NBUF:]
    sid = lax.axis_index("s")
    wid = sid * NC + lax.axis_index("c")
    n_my = jnp.where(wid < EXTRA, BASE_CHUNKS + 1, BASE_CHUNKS)
    start_chunk = wid * BASE_CHUNKS + jnp.minimum(wid, EXTRA)
    atom0 = start_chunk * CHUNK

    @pl.when(sid == 0)
    def _stage_table():
        pltpu.sync_copy(table_hbm, table_sh)

    pltpu.sync_copy(
        idx_hbm.at[pl.ds(atom0, BASE_CHUNKS * CHUNK)],
        idx_v.at[pl.ds(0, BASE_CHUNKS * CHUNK)],
    )

    @pl.when(wid < EXTRA)
    def _extra_idx():
        pltpu.sync_copy(
            idx_hbm.at[pl.ds(atom0 + BASE_CHUNKS * CHUNK, CHUNK)],
            idx_v.at[pl.ds(BASE_CHUNKS * CHUNK, CHUNK)],
        )

    @pl.when(wid == NW - 1)
    def _tail_idx():
        pltpu.sync_copy(
            idx_hbm.at[pl.ds(N_FULL * CHUNK, TAIL)],
            idx_v.at[pl.ds(BASE_CHUNKS * CHUNK, TAIL)],
        )

    plsc.subcore_barrier()

    def _gather(v, b):
        return pltpu.make_async_copy(
            table_sh.at[idx_v.at[pl.ds(v * CHUNK, CHUNK)]], rows_v.at[b], gsem[b]
        )

    def _store(v, b):
        return pltpu.make_async_copy(
            rows_v.at[b], out_hbm.at[pl.ds((start_chunk + v) * CHUNK, CHUNK)], ssem[b]
        )

    def round_body(r, carry):
        for b in range(NBUF):
            v = r * NBUF + b

            @pl.when(v < n_my)
            def _(b=b, v=v):
                @pl.when(r >= 1)
                def _wait_prev():
                    _store(v - NBUF, b).wait()

                _gather(v, b).start()

        for b in range(NBUF):
            v = r * NBUF + b

            @pl.when(v < n_my)
            def _(b=b, v=v):
                _gather(v, b).wait()
                _store(v, b).start()

        return carry

    lax.fori_loop(0, ROUNDS, round_body, 0)

    for b in range(NBUF):
        v = (ROUNDS - 1) * NBUF + b
        vlast = jnp.where(v < n_my, v, v - NBUF)

        @pl.when((vlast >= 0) & (vlast < n_my))
        def _(b=b, vlast=vlast):
            _store(vlast, b).wait()

    @pl.when(wid == NW - 1)
    def _tail():
        base = N_FULL * CHUNK
        idx_t = idx_v.at[pl.ds(BASE_CHUNKS * CHUNK, TAIL)]
        rows_t = rows_v.at[0].at[pl.ds(0, TAIL)]
        pltpu.async_copy(table_sh.at[idx_t], rows_t, sem).wait()
        pltpu.sync_copy(rows_t, out_hbm.at[pl.ds(base, TAIL)])


def kernel(Z, embeddings):
    return _embed(Z.astype(jnp.int32), embeddings)
